# Initial kernel scaffold; baseline (speedup 1.0000x reference)
import jax
import jax.numpy as jnp
from jax.experimental import pallas as pl


def kernel(x_mark, minute_table, hour_table):
    raise NotImplementedError("write your pallas kernel here")


# speedup vs baseline: 2.9336x; 2.9336x over previous
import functools

import jax
import jax.numpy as jnp
import numpy as np
from jax import lax
from jax.experimental import pallas as pl
from jax.experimental.pallas import tpu as pltpu
from jax.experimental.pallas import tpu_sc as plsc

D_MODEL = 512
N_MIN = 60
N_HR = 24
SEQ_OUT = 35
_IDX35 = np.linspace(0, 49, SEQ_OUT).astype(np.int32)

NC, NS = 2, 16
NW = NC * NS
B_ROWS = 4096 * SEQ_OUT
BPW = B_ROWS // NW
CHUNK = 64
NCHUNK = BPW // CHUNK


def _combine_body(m_ref, h_ref, out_ref):
    out_ref[...] = m_ref[...][:, None, :] + h_ref[...][None, :, :]


def _combined_table(minute_table, hour_table):
    return pl.pallas_call(
        _combine_body,
        out_shape=jax.ShapeDtypeStruct((N_MIN, N_HR, D_MODEL), jnp.float32),
    )(minute_table, hour_table)


def _sc_body(comb_hbm, cidx_hbm, out_hbm, idx_v, buf_v, g0, g1, s0, s1):
    gsem = (g0, g1)
    ssem = (s0, s1)
    wid = lax.axis_index("s") * NC + lax.axis_index("c")
    base = wid * BPW
    pltpu.sync_copy(cidx_hbm.at[pl.ds(base, BPW)], idx_v)

    def start_gather(g):
        pltpu.async_copy(
            comb_hbm.at[idx_v.at[pl.ds(g * CHUNK, CHUNK)]],
            buf_v.at[g % 2],
            gsem[g % 2],
        )

    def wait_gather(g):
        pltpu.make_async_copy(
            comb_hbm.at[idx_v.at[pl.ds(g * CHUNK, CHUNK)]],
            buf_v.at[g % 2],
            gsem[g % 2],
        ).wait()

    def start_scatter(g):
        pltpu.async_copy(
            buf_v.at[g % 2],
            out_hbm.at[pl.ds(base + g * CHUNK, CHUNK)],
            ssem[g % 2],
        )

    def wait_scatter(g):
        pltpu.make_async_copy(
            buf_v.at[g % 2],
            out_hbm.at[pl.ds(base + g * CHUNK, CHUNK)],
            ssem[g % 2],
        ).wait()

    start_gather(0)
    for g in range(NCHUNK):
        if g + 1 < NCHUNK:
            if g >= 1:
                wait_scatter(g - 1)
            start_gather(g + 1)
        wait_gather(g)
        start_scatter(g)
    wait_scatter(NCHUNK - 2)
    wait_scatter(NCHUNK - 1)


_sc_gather = functools.partial(
    pl.kernel,
    out_type=jax.ShapeDtypeStruct((B_ROWS, D_MODEL), jnp.float32),
    mesh=plsc.VectorSubcoreMesh(core_axis_name="c", subcore_axis_name="s"),
    scratch_types=[
        pltpu.VMEM((BPW,), jnp.int32),
        pltpu.VMEM((2, CHUNK, D_MODEL), jnp.float32),
        pltpu.SemaphoreType.DMA,
        pltpu.SemaphoreType.DMA,
        pltpu.SemaphoreType.DMA,
        pltpu.SemaphoreType.DMA,
    ],
)(_sc_body)


def kernel(x_mark, minute_table, hour_table):
    xs = x_mark[:, _IDX35, :].astype(jnp.int32)
    cidx = (xs[..., 0] * N_HR + xs[..., 1]).reshape(B_ROWS)
    comb = _combined_table(minute_table, hour_table).reshape(N_MIN * N_HR, D_MODEL)
    out = _sc_gather(comb, cidx)
    return out.reshape(4096, SEQ_OUT, D_MODEL)
